# Initial kernel scaffold; baseline (speedup 1.0000x reference)
import jax
import jax.numpy as jnp
from jax.experimental import pallas as pl


def kernel(point_coordinates, keypoint_indices, set_indices, pW0, pb0, pg0, pbeta0, pW1, pb1, pg1, pbeta1, pW2, pb2, pg2, pbeta2, pW3, pb3, pg3, pbeta3, oW0, ob0, og0, obeta0, oW1, ob1, og1, obeta1):
    raise NotImplementedError("write your pallas kernel here")


# speedup vs baseline: 1.4241x; 1.4241x over previous
import jax
import jax.numpy as jnp
from jax.experimental import pallas as pl

_EPS = 1e-5


def _pt_layer_kernel(x_ref, w_ref, b_ref, g_ref, be_ref, mv_ref, y_ref, s_ref):
    i = pl.program_id(0)
    x = x_ref[...]
    m = mv_ref[0:1, :]
    v = mv_ref[1:2, :]
    xn = g_ref[...] * (x - m) * jax.lax.rsqrt(v + _EPS) + be_ref[...]
    y = jnp.dot(xn, w_ref[...], preferred_element_type=jnp.float32)
    y = jnp.maximum(y + b_ref[...], 0.0)
    y_ref[...] = y

    @pl.when(i == 0)
    def _():
        s_ref[...] = jnp.zeros_like(s_ref)

    upd = jnp.concatenate(
        [jnp.sum(y, axis=0, keepdims=True),
         jnp.sum(y * y, axis=0, keepdims=True)], axis=0)
    s_ref[...] = s_ref[...] + upd


def _pt_layer(x, w, b, g, be, mv, block_rows):
    s_rows, din = x.shape
    dout = w.shape[1]
    grid = s_rows // block_rows
    y, sums = pl.pallas_call(
        _pt_layer_kernel,
        grid=(grid,),
        in_specs=[
            pl.BlockSpec((block_rows, din), lambda i: (i, 0)),
            pl.BlockSpec((din, dout), lambda i: (0, 0)),
            pl.BlockSpec((1, dout), lambda i: (0, 0)),
            pl.BlockSpec((1, din), lambda i: (0, 0)),
            pl.BlockSpec((1, din), lambda i: (0, 0)),
            pl.BlockSpec((2, din), lambda i: (0, 0)),
        ],
        out_specs=[
            pl.BlockSpec((block_rows, dout), lambda i: (i, 0)),
            pl.BlockSpec((2, dout), lambda i: (0, 0)),
        ],
        out_shape=[
            jax.ShapeDtypeStruct((s_rows, dout), jnp.float32),
            jax.ShapeDtypeStruct((2, dout), jnp.float32),
        ],
    )(x, w, b.reshape(1, -1), g.reshape(1, -1), be.reshape(1, -1), mv)
    mean = sums[0] / s_rows
    var = sums[1] / s_rows - mean * mean
    return y, jnp.stack([mean, var], axis=0)


def _bn_in_kernel(z, g, be):
    m = jnp.mean(z, axis=0, keepdims=True)
    v = jnp.mean((z - m) * (z - m), axis=0, keepdims=True)
    return g * (z - m) * jax.lax.rsqrt(v + _EPS) + be


def _out_kernel(x_ref, mv_ref, g3_ref, be3_ref,
                w0_ref, b0_ref, g0_ref, be0_ref,
                w1_ref, b1_ref, g1_ref, be1_ref, o_ref):
    raw = x_ref[...]
    m = mv_ref[0:1, :]
    v = mv_ref[1:2, :]
    feat = g3_ref[...] * (raw - m) * jax.lax.rsqrt(v + _EPS) + be3_ref[...]
    feat = jnp.where(jnp.isneginf(feat), 0.0, feat)
    z = jnp.dot(feat, w0_ref[...], preferred_element_type=jnp.float32)
    z = jnp.maximum(z + b0_ref[...], 0.0)
    z = _bn_in_kernel(z, g0_ref[...], be0_ref[...])
    z = jnp.dot(z, w1_ref[...], preferred_element_type=jnp.float32)
    z = jnp.maximum(z + b1_ref[...], 0.0)
    o_ref[...] = _bn_in_kernel(z, g1_ref[...], be1_ref[...])


def kernel(point_coordinates, keypoint_indices, set_indices,
           pW0, pb0, pg0, pbeta0, pW1, pb1, pg1, pbeta1,
           pW2, pb2, pg2, pbeta2, pW3, pb3, pg3, pbeta3,
           oW0, ob0, og0, obeta0, oW1, ob1, og1, obeta1):
    s_rows = set_indices.shape[0]
    k_rows = keypoint_indices.shape[0]

    kp_coords = jnp.take(point_coordinates, keypoint_indices[:, 0], axis=0)
    src = jnp.take(point_coordinates, set_indices[:, 0], axis=0)
    dst = jnp.take(kp_coords, set_indices[:, 1], axis=0)
    disp = src - dst

    din0 = disp.shape[1]
    id_mv = jnp.stack([jnp.zeros((din0,), jnp.float32),
                       jnp.full((din0,), 1.0 - _EPS, jnp.float32)], axis=0)
    ones = jnp.ones((din0,), jnp.float32)
    zeros = jnp.zeros((din0,), jnp.float32)

    block = 5000
    x, mv = _pt_layer(disp, pW0, pb0, ones, zeros, id_mv, block)
    x, mv = _pt_layer(x, pW1, pb1, pg0, pbeta0, mv, block)
    x, mv = _pt_layer(x, pW2, pb2, pg1, pbeta1, mv, block)
    x, mv = _pt_layer(x, pW3, pb3, pg2, pbeta2, mv, block)

    seg_raw = jax.ops.segment_max(x, set_indices[:, 1], num_segments=k_rows)

    dout = oW1.shape[1]
    out = pl.pallas_call(
        _out_kernel,
        in_specs=[pl.BlockSpec(seg_raw.shape, lambda: (0, 0)),
                  pl.BlockSpec((2, 300), lambda: (0, 0)),
                  pl.BlockSpec((1, 300), lambda: (0, 0)),
                  pl.BlockSpec((1, 300), lambda: (0, 0)),
                  pl.BlockSpec(oW0.shape, lambda: (0, 0)),
                  pl.BlockSpec((1, 300), lambda: (0, 0)),
                  pl.BlockSpec((1, 300), lambda: (0, 0)),
                  pl.BlockSpec((1, 300), lambda: (0, 0)),
                  pl.BlockSpec(oW1.shape, lambda: (0, 0)),
                  pl.BlockSpec((1, dout), lambda: (0, 0)),
                  pl.BlockSpec((1, dout), lambda: (0, 0)),
                  pl.BlockSpec((1, dout), lambda: (0, 0))],
        out_specs=pl.BlockSpec((k_rows, dout), lambda: (0, 0)),
        out_shape=jax.ShapeDtypeStruct((k_rows, dout), jnp.float32),
    )(seg_raw, mv, pg3.reshape(1, -1), pbeta3.reshape(1, -1),
      oW0, ob0.reshape(1, -1), og0.reshape(1, -1), obeta0.reshape(1, -1),
      oW1, ob1.reshape(1, -1), og1.reshape(1, -1), obeta1.reshape(1, -1))
    return out
